# initial kernel scaffold (unmeasured)
import jax
import jax.numpy as jnp
from jax import lax
from jax.experimental import pallas as pl
from jax.experimental.pallas import tpu as pltpu

N_DEV = 4
T_LOC = 128
D = 512
E = 8
E_LOC = 2
F = 1024
T = N_DEV * T_LOC


def kernel(x, router, W1, W2):
    assert x.shape == (T_LOC, D)
    assert router.shape == (T, E_LOC)
    assert W1.shape == (E_LOC, D, F)
    assert W2.shape == (E_LOC, F, D)

    def body(
        x_ref, r_ref, w1_ref, w2_ref, out_ref,
        xg_ref,
        rg_ref,
        pr_ref,
        rs_stage_ref,
        rs_recv_ref,
        agx_ss, agx_rs, agr_ss, agr_rs, rs_ss, rs_rs,
    ):
        my = lax.axis_index("i")
        left = (my + N_DEV - 1) % N_DEV
        right = (my + 1) % N_DEV

        barrier = pltpu.get_barrier_semaphore()
        for nbr in (left, right):
            pl.semaphore_signal(
                barrier, inc=1,
                device_id=(nbr,), device_id_type=pl.DeviceIdType.MESH,
            )
        pl.semaphore_wait(barrier, 2)

        pl.store(xg_ref, (pl.ds(my * T_LOC, T_LOC), slice(None)), x_ref[...])
        pl.store(
            rg_ref,
            (pl.ds(my, 1), slice(None), slice(None)),
            r_ref[...][None, :, :],
        )
        for h in range(N_DEV - 1):
            cs = (my - h + N_DEV) % N_DEV
            rdma_x = pltpu.make_async_remote_copy(
                src_ref=xg_ref.at[pl.ds(cs * T_LOC, T_LOC), :],
                dst_ref=xg_ref.at[pl.ds(cs * T_LOC, T_LOC), :],
                send_sem=agx_ss.at[h],
                recv_sem=agx_rs.at[h],
                device_id=(right,),
                device_id_type=pl.DeviceIdType.MESH,
            )
            rdma_r = pltpu.make_async_remote_copy(
                src_ref=rg_ref.at[pl.ds(cs, 1)],
                dst_ref=rg_ref.at[pl.ds(cs, 1)],
                send_sem=agr_ss.at[h],
                recv_sem=agr_rs.at[h],
                device_id=(right,),
                device_id_type=pl.DeviceIdType.MESH,
            )
            rdma_x.start()
            rdma_r.start()
            rdma_x.wait()
            rdma_r.wait()

        xgv = xg_ref[...]
        rgv = rg_ref[...]
        router_full = jnp.concatenate(
            [rgv[o] for o in range(N_DEV)], axis=1
        )
        gates = jnp.dot(
            xgv, router_full, precision=lax.Precision.HIGHEST
        )

        ids = lax.broadcasted_iota(jnp.int32, (T, E), 1)
        m1 = jnp.max(gates, axis=1, keepdims=True)
        i1 = jnp.min(jnp.where(gates == m1, ids, E), axis=1, keepdims=True)
        oh1 = ids == i1
        g2 = jnp.where(oh1, jnp.float32(-1e30), gates)
        m2 = jnp.max(g2, axis=1, keepdims=True)
        i2 = jnp.min(jnp.where(g2 == m2, ids, E), axis=1, keepdims=True)
        oh2 = ids == i2
        e2 = jnp.exp(m2 - m1)
        w_top1 = 1.0 / (1.0 + e2)
        w_top2 = e2 / (1.0 + e2)
        wdense = (
            jnp.where(oh1, w_top1, 0.0) + jnp.where(oh2, w_top2, 0.0)
        )

        partial = jnp.zeros((T, D), jnp.float32)
        for k in range(E_LOC):
            ge = E_LOC * my + k
            wcol = jnp.sum(
                jnp.where(ids == ge, wdense, 0.0), axis=1, keepdims=True
            )
            hdn = jnp.maximum(
                jnp.dot(xgv, w1_ref[k], preferred_element_type=jnp.float32),
                0.0,
            )
            partial = partial + jnp.dot(
                hdn, w2_ref[k], preferred_element_type=jnp.float32
            ) * wcol
        pr_ref[...] = partial

        for s in range(N_DEV - 1):
            c = (my + 1 + s) % N_DEV
            chunk = pl.load(pr_ref, (pl.ds(c * T_LOC, T_LOC), slice(None)))
            if s > 0:
                chunk = chunk + rs_recv_ref[s - 1]
            rs_stage_ref[s] = chunk
            rdma = pltpu.make_async_remote_copy(
                src_ref=rs_stage_ref.at[s],
                dst_ref=rs_recv_ref.at[s],
                send_sem=rs_ss.at[s],
                recv_sem=rs_rs.at[s],
                device_id=(left,),
                device_id_type=pl.DeviceIdType.MESH,
            )
            rdma.start()
            rdma.wait()

        out_ref[...] = (
            pl.load(pr_ref, (pl.ds(my * T_LOC, T_LOC), slice(None)))
            + rs_recv_ref[N_DEV - 2]
        )

    return pl.pallas_call(
        body,
        out_shape=jax.ShapeDtypeStruct((T_LOC, D), jnp.float32),
        in_specs=[
            pl.BlockSpec(memory_space=pltpu.VMEM),
            pl.BlockSpec(memory_space=pltpu.VMEM),
            pl.BlockSpec(memory_space=pltpu.VMEM),
            pl.BlockSpec(memory_space=pltpu.VMEM),
        ],
        out_specs=pl.BlockSpec(memory_space=pltpu.VMEM),
        scratch_shapes=[
            pltpu.VMEM((T, D), jnp.float32),
            pltpu.VMEM((N_DEV, T, E_LOC), jnp.float32),
            pltpu.VMEM((T, D), jnp.float32),
            pltpu.VMEM((N_DEV - 1, T_LOC, D), jnp.float32),
            pltpu.VMEM((N_DEV - 1, T_LOC, D), jnp.float32),
            pltpu.SemaphoreType.DMA((N_DEV - 1,)),
            pltpu.SemaphoreType.DMA((N_DEV - 1,)),
            pltpu.SemaphoreType.DMA((N_DEV - 1,)),
            pltpu.SemaphoreType.DMA((N_DEV - 1,)),
            pltpu.SemaphoreType.DMA((N_DEV - 1,)),
            pltpu.SemaphoreType.DMA((N_DEV - 1,)),
        ],
        compiler_params=pltpu.CompilerParams(collective_id=0),
    )(x, router, W1, W2)


# baseline (device time: 48948 ns/iter reference)
import jax
import jax.numpy as jnp
from jax import lax
from jax.experimental import pallas as pl
from jax.experimental.pallas import tpu as pltpu

N_DEV = 4
T_LOC = 128
D = 512
E = 8
E_LOC = 2
F = 1024
T = N_DEV * T_LOC


def kernel(x, router, W1, W2):
    assert x.shape == (T_LOC, D)
    assert router.shape == (T, E_LOC)
    assert W1.shape == (E_LOC, D, F)
    assert W2.shape == (E_LOC, F, D)

    def body(
        x_ref, r_ref, w1_ref, w2_ref, out_ref,
        xg_ref,
        rg_ref,
        pr_ref,
        rs_stage_ref,
        rs_recv_ref,
        agx_ss, agx_rs, agr_ss, agr_rs, rs_ss, rs_rs,
    ):
        my = lax.axis_index("i")
        left = (my + N_DEV - 1) % N_DEV
        right = (my + 1) % N_DEV

        barrier = pltpu.get_barrier_semaphore()
        for nbr in (left, right):
            pl.semaphore_signal(
                barrier, inc=1,
                device_id=(nbr,), device_id_type=pl.DeviceIdType.MESH,
            )
        pl.semaphore_wait(barrier, 2)

        xg_ref[pl.ds(my * T_LOC, T_LOC), :] = x_ref[...]
        rg_ref[pl.ds(my, 1), :, :] = r_ref[...][None, :, :]
        for h in range(N_DEV - 1):
            cs = (my - h + N_DEV) % N_DEV
            rdma_x = pltpu.make_async_remote_copy(
                src_ref=xg_ref.at[pl.ds(cs * T_LOC, T_LOC), :],
                dst_ref=xg_ref.at[pl.ds(cs * T_LOC, T_LOC), :],
                send_sem=agx_ss.at[h],
                recv_sem=agx_rs.at[h],
                device_id=(right,),
                device_id_type=pl.DeviceIdType.MESH,
            )
            rdma_r = pltpu.make_async_remote_copy(
                src_ref=rg_ref.at[pl.ds(cs, 1)],
                dst_ref=rg_ref.at[pl.ds(cs, 1)],
                send_sem=agr_ss.at[h],
                recv_sem=agr_rs.at[h],
                device_id=(right,),
                device_id_type=pl.DeviceIdType.MESH,
            )
            rdma_x.start()
            rdma_r.start()
            rdma_x.wait()
            rdma_r.wait()

        xgv = xg_ref[...]
        rgv = rg_ref[...]
        router_full = jnp.concatenate(
            [rgv[o] for o in range(N_DEV)], axis=1
        )
        gates = jnp.dot(
            xgv, router_full, precision=lax.Precision.HIGHEST
        )

        ids = lax.broadcasted_iota(jnp.int32, (T, E), 1)
        m1 = jnp.max(gates, axis=1, keepdims=True)
        i1 = jnp.min(jnp.where(gates == m1, ids, E), axis=1, keepdims=True)
        oh1 = ids == i1
        g2 = jnp.where(oh1, jnp.float32(-1e30), gates)
        m2 = jnp.max(g2, axis=1, keepdims=True)
        i2 = jnp.min(jnp.where(g2 == m2, ids, E), axis=1, keepdims=True)
        oh2 = ids == i2
        e2 = jnp.exp(m2 - m1)
        w_top1 = 1.0 / (1.0 + e2)
        w_top2 = e2 / (1.0 + e2)
        wdense = (
            jnp.where(oh1, w_top1, 0.0) + jnp.where(oh2, w_top2, 0.0)
        )

        partial = jnp.zeros((T, D), jnp.float32)
        for k in range(E_LOC):
            ge = E_LOC * my + k
            wcol = jnp.sum(
                jnp.where(ids == ge, wdense, 0.0), axis=1, keepdims=True
            )
            hdn = jnp.maximum(
                jnp.dot(xgv, w1_ref[k], preferred_element_type=jnp.float32),
                0.0,
            )
            partial = partial + jnp.dot(
                hdn, w2_ref[k], preferred_element_type=jnp.float32
            ) * wcol
        pr_ref[...] = partial

        for s in range(N_DEV - 1):
            c = (my + 1 + s) % N_DEV
            chunk = pr_ref[pl.ds(c * T_LOC, T_LOC), :]
            if s > 0:
                chunk = chunk + rs_recv_ref[s - 1]
            rs_stage_ref[s] = chunk
            rdma = pltpu.make_async_remote_copy(
                src_ref=rs_stage_ref.at[s],
                dst_ref=rs_recv_ref.at[s],
                send_sem=rs_ss.at[s],
                recv_sem=rs_rs.at[s],
                device_id=(left,),
                device_id_type=pl.DeviceIdType.MESH,
            )
            rdma.start()
            rdma.wait()

        out_ref[...] = (
            pr_ref[pl.ds(my * T_LOC, T_LOC), :] + rs_recv_ref[N_DEV - 2]
        )

    return pl.pallas_call(
        body,
        out_shape=jax.ShapeDtypeStruct((T_LOC, D), jnp.float32),
        in_specs=[
            pl.BlockSpec(memory_space=pltpu.VMEM),
            pl.BlockSpec(memory_space=pltpu.VMEM),
            pl.BlockSpec(memory_space=pltpu.VMEM),
            pl.BlockSpec(memory_space=pltpu.VMEM),
        ],
        out_specs=pl.BlockSpec(memory_space=pltpu.VMEM),
        scratch_shapes=[
            pltpu.VMEM((T, D), jnp.float32),
            pltpu.VMEM((N_DEV, T, E_LOC), jnp.float32),
            pltpu.VMEM((T, D), jnp.float32),
            pltpu.VMEM((N_DEV - 1, T_LOC, D), jnp.float32),
            pltpu.VMEM((N_DEV - 1, T_LOC, D), jnp.float32),
            pltpu.SemaphoreType.DMA((N_DEV - 1,)),
            pltpu.SemaphoreType.DMA((N_DEV - 1,)),
            pltpu.SemaphoreType.DMA((N_DEV - 1,)),
            pltpu.SemaphoreType.DMA((N_DEV - 1,)),
            pltpu.SemaphoreType.DMA((N_DEV - 1,)),
            pltpu.SemaphoreType.DMA((N_DEV - 1,)),
        ],
        compiler_params=pltpu.CompilerParams(collective_id=0),
    )(x, router, W1, W2)
